# baseline (speedup 1.0000x reference)
import functools

import jax
import jax.numpy as jnp
from jax import lax
from jax.experimental import pallas as pl
from jax.experimental.pallas import tpu as pltpu
from jax.experimental.pallas import tpu_sc as plsc

_VOCAB = 100000
_EMBED = 64
_BATCH = 4096
_CTX = 20

_EAUG = 72
_SHIFT = 16.0

_NC = 2
_NS = 16
_NW = _NC * _NS
_RPW = _BATCH // _NW
_CH = 64
_NCH = _RPW // _CH

_BB = 1024
_NB = _BATCH // _BB
_VT = 2048
_NV = 50
_VPAD = _NV * _VT

_OB = 128
_ONB = _BATCH // _OB
_OVT = 4096
_ONMAIN = 24
_OTAIL = _VOCAB - _ONMAIN * _OVT
_ONV = _ONMAIN + 1
_NBUF = 8
_NTBUF = 2


def _sc_embed_sigmoid(x_flat, emb):
  mesh = plsc.VectorSubcoreMesh(core_axis_name="c", subcore_axis_name="s")

  @functools.partial(
      pl.kernel,
      mesh=mesh,
      out_type=jax.ShapeDtypeStruct((_BATCH, _EMBED), jnp.float32),
      compiler_params=pltpu.CompilerParams(use_tc_tiling_on_sc=False),
      scratch_types=[
          pltpu.VMEM((_RPW * _CTX,), jnp.int32),
          pltpu.VMEM((_CH * _CTX, _EMBED), jnp.float32),
          pltpu.VMEM((_RPW, _EMBED), jnp.float32),
          pltpu.SemaphoreType.DMA,
      ],
  )
  def k(x_hbm, emb_hbm, out_hbm, idx_v, rows_v, out_v, sem):
    wid = lax.axis_index("s") * _NC + lax.axis_index("c")
    base = wid * _RPW
    pltpu.sync_copy(x_hbm.at[pl.ds(base * _CTX, _RPW * _CTX)], idx_v)
    for c in range(_NCH):
      pltpu.async_copy(
          emb_hbm.at[idx_v.at[pl.ds(c * _CH * _CTX, _CH * _CTX)]],
          rows_v, sem).wait()

      def row(i, _, c=c):
        for l in range(_EMBED // 16):
          sl = pl.ds(l * 16, 16)
          acc = rows_v[i * _CTX, sl]
          for j in range(1, _CTX):
            acc = acc + rows_v[i * _CTX + j, sl]
          out_v[c * _CH + i, sl] = 1.0 / (1.0 + jnp.exp(-acc))
        return 0

      lax.fori_loop(0, _CH, row, 0)
    pltpu.sync_copy(out_v, out_hbm.at[pl.ds(base, _RPW)])

  return k(x_flat, emb)


def _logs_body(sig_ref, w_ref, logs_ref, s_acc):
  v = pl.program_id(1)

  @pl.when(v == 0)
  def _():
    s_acc[...] = jnp.zeros(s_acc.shape, jnp.float32)

  l16 = lax.dot_general(sig_ref[...], w_ref[...], (((1,), (1,)), ((), ())),
                        preferred_element_type=jnp.float32)
  s_acc[...] = s_acc[...] + jnp.sum(jnp.exp(l16), axis=1, keepdims=True)

  @pl.when(v == _NV - 1)
  def _():
    logs_ref[...] = jnp.log(s_acc[...])


def _main_dst(out_hbm, cnt):
  pi = cnt // _ONMAIN
  pj = lax.rem(cnt, _ONMAIN)
  return out_hbm.at[pl.ds(pi * _OB, _OB), pl.ds(pj * _OVT, _OVT)]


def _out_body(sig_ref, w_ref, out_hbm, buf, tbuf, sems, tsems):
  i = pl.program_id(0)
  j = pl.program_id(1)
  val = lax.dot_general(sig_ref[...], w_ref[...], (((1,), (1,)), ((), ())),
                        preferred_element_type=jnp.float32)

  @pl.when(j < _ONMAIN)
  def _():
    cnt = i * _ONMAIN + j
    k = lax.rem(cnt, _NBUF)

    @pl.when(cnt >= _NBUF)
    def _():
      pltpu.make_async_copy(buf.at[k], _main_dst(out_hbm, cnt - _NBUF),
                            sems.at[k]).wait()

    buf[k] = val
    pltpu.make_async_copy(buf.at[k], _main_dst(out_hbm, cnt),
                          sems.at[k]).start()

  @pl.when(j == _ONMAIN)
  def _():
    kt = lax.rem(i, _NTBUF)

    @pl.when(i >= _NTBUF)
    def _():
      pltpu.make_async_copy(
          tbuf.at[kt],
          out_hbm.at[pl.ds((i - _NTBUF) * _OB, _OB),
                     pl.ds(_ONMAIN * _OVT, _OTAIL)],
          tsems.at[kt]).wait()

    tbuf[kt] = val[:, :_OTAIL]
    pltpu.make_async_copy(
        tbuf.at[kt],
        out_hbm.at[pl.ds(i * _OB, _OB), pl.ds(_ONMAIN * _OVT, _OTAIL)],
        tsems.at[kt]).start()

  @pl.when((i == _ONB - 1) & (j == _ONMAIN))
  def _():
    total = _ONB * _ONMAIN
    for d in range(_NBUF):
      cnt = total - _NBUF + d
      pltpu.make_async_copy(buf.at[cnt % _NBUF], _main_dst(out_hbm, cnt),
                            sems.at[cnt % _NBUF]).wait()
    for d in range(_NTBUF):
      ii = _ONB - _NTBUF + d
      pltpu.make_async_copy(
          tbuf.at[ii % _NTBUF],
          out_hbm.at[pl.ds(ii * _OB, _OB),
                     pl.ds(_ONMAIN * _OVT, _OTAIL)],
          tsems.at[ii % _NTBUF]).wait()


def kernel(x, emb, W, b):
  sig = _sc_embed_sigmoid(x.reshape(-1).astype(jnp.int32), emb)

  w2 = jnp.zeros((_VPAD, _EAUG), jnp.float32)
  w2 = w2.at[:_VOCAB, :_EMBED].set(W)
  w2 = w2.at[:, _EMBED].set(jnp.pad(b, (0, _VPAD - _VOCAB),
                                    constant_values=-1e30))
  w2 = w2.at[:, _EMBED + 1].set(-_SHIFT)
  w2 = w2.at[:, _EMBED + 2].set(-1.0)
  w2 = w2.astype(jnp.bfloat16)

  one = jnp.ones((_BATCH, 1), jnp.float32)
  zero5 = jnp.zeros((_BATCH, _EAUG - _EMBED - 3), jnp.float32)
  sig_logs = jnp.concatenate(
      [sig, one, one, jnp.zeros((_BATCH, 1), jnp.float32), zero5],
      axis=1).astype(jnp.bfloat16)

  logs = pl.pallas_call(
      _logs_body,
      grid=(_NB, _NV),
      in_specs=[
          pl.BlockSpec((_BB, _EAUG), lambda i, j: (i, 0)),
          pl.BlockSpec((_VT, _EAUG), lambda i, j: (j, 0)),
      ],
      out_specs=pl.BlockSpec((_BB, 1), lambda i, j: (i, 0)),
      out_shape=jax.ShapeDtypeStruct((_BATCH, 1), jnp.float32),
      scratch_shapes=[pltpu.VMEM((_BB, 1), jnp.float32)],
  )(sig_logs, w2)

  sig_out = jnp.concatenate([sig, one, one, logs, zero5],
                            axis=1).astype(jnp.bfloat16)

  out = pl.pallas_call(
      _out_body,
      grid=(_ONB, _ONV),
      in_specs=[
          pl.BlockSpec((_OB, _EAUG), lambda i, j: (i, 0)),
          pl.BlockSpec((_OVT, _EAUG), lambda i, j: (j, 0)),
      ],
      out_specs=pl.BlockSpec(memory_space=pl.ANY),
      out_shape=jax.ShapeDtypeStruct((_BATCH, _VOCAB), jnp.float32),
      scratch_shapes=[
          pltpu.VMEM((_NBUF, _OB, _OVT), jnp.float32),
          pltpu.VMEM((_NTBUF, _OB, _OTAIL), jnp.float32),
          pltpu.SemaphoreType.DMA((_NBUF,)),
          pltpu.SemaphoreType.DMA((_NTBUF,)),
      ],
  )(sig_out, w2)
  return out
